# Initial kernel scaffold; baseline (speedup 1.0000x reference)
import jax
import jax.numpy as jnp
from jax.experimental import pallas as pl


def kernel(frame1, frame2, depth1, flow12):
    raise NotImplementedError("write your pallas kernel here")


# speedup vs baseline: 1.0460x; 1.0460x over previous
import jax
import jax.numpy as jnp
from jax import lax
from jax.experimental import pallas as pl

B, C, H, W = 2, 3, 1080, 1920
HP = 1096
WP = 1922
NCELL = B * HP * WP
ROWS_BLK = 8
NBLK = B * H // ROWS_BLK
NROWS = B * H
N = B * H * W


def _logmax_body(depth_ref, out_ref):
    d = depth_ref[0, 0]
    out_ref[...] = jnp.max(jnp.log1p(jnp.clip(d, 0.0, 1000.0))).reshape(1, 1, 1)


def _logmax(depth1):
    return pl.pallas_call(
        _logmax_body,
        grid=(NBLK,),
        in_specs=[pl.BlockSpec((1, 1, ROWS_BLK, W),
                               lambda k: (k // (H // ROWS_BLK), 0, k % (H // ROWS_BLK), 0))],
        out_specs=pl.BlockSpec((1, 1, 1), lambda k: (k, 0, 0)),
        out_shape=jax.ShapeDtypeStruct((NBLK, 1, 1), jnp.float32),
    )(depth1)


def _records_body(scale_ref, frame_ref, depth_ref, flow_ref, idx_ref, vals_ref):
    scale = scale_ref[0, 0]
    i = pl.program_id(1)
    bprog = pl.program_id(0)
    fl = flow_ref[0]
    gx = lax.broadcasted_iota(jnp.int32, (ROWS_BLK, W), 1).astype(jnp.float32)
    gy = lax.broadcasted_iota(jnp.int32, (ROWS_BLK, W), 0).astype(jnp.float32)
    tpox = fl[0] + gx + 1.0
    tpoy = fl[1] + gy + ((i * ROWS_BLK).astype(jnp.float32) + 1.0)
    fxi = jnp.clip(jnp.floor(tpox).astype(jnp.int32), 0, W + 1)
    cxi = jnp.clip(jnp.ceil(tpox).astype(jnp.int32), 0, W + 1)
    fyi = jnp.clip(jnp.floor(tpoy).astype(jnp.int32), 0, H + 1)
    cyi = jnp.clip(jnp.ceil(tpoy).astype(jnp.int32), 0, H + 1)
    fracx = jnp.clip(tpox, 0.0, float(W + 1)) - fxi.astype(jnp.float32)
    fracy = jnp.clip(tpoy, 0.0, float(H + 1)) - fyi.astype(jnp.float32)
    d = depth_ref[0, 0]
    invd = jnp.exp(jnp.log1p(jnp.clip(d, 0.0, 1000.0)) * (-scale))
    u0 = (1.0 - fracy) * invd
    u1 = fracy * invd
    v0 = 1.0 - fracx
    v1 = fracx
    fr = frame_ref[0]
    it = (bprog * HP + fyi + 7) * WP + fxi
    ib = it + (cyi - fyi) * WP
    w00 = u0 * v0
    w10 = u1 * v0
    w01 = u0 * v1
    w11 = u1 * v1
    idx_ref[...] = jnp.stack([it, it + 1, ib, ib + 1], axis=1)
    vals_ref[...] = jnp.stack(
        [w00 * fr[0], w00 * fr[1], w00 * fr[2], w00,
         w01 * fr[0], w01 * fr[1], w01 * fr[2], w01,
         w10 * fr[0], w10 * fr[1], w10 * fr[2], w10,
         w11 * fr[0], w11 * fr[1], w11 * fr[2], w11], axis=1)


def _records(scale_arr, frame1, depth1, flow12):
    nby = H // ROWS_BLK
    return pl.pallas_call(
        _records_body,
        grid=(B, nby),
        in_specs=[
            pl.BlockSpec((1, 1), lambda b, i: (0, 0)),
            pl.BlockSpec((1, 3, ROWS_BLK, W), lambda b, i: (b, 0, i, 0)),
            pl.BlockSpec((1, 1, ROWS_BLK, W), lambda b, i: (b, 0, i, 0)),
            pl.BlockSpec((1, 2, ROWS_BLK, W), lambda b, i: (b, 0, i, 0)),
        ],
        out_specs=[
            pl.BlockSpec((ROWS_BLK, 4, W), lambda b, i: (b * nby + i, 0, 0)),
            pl.BlockSpec((ROWS_BLK, 16, W), lambda b, i: (b * nby + i, 0, 0)),
        ],
        out_shape=[
            jax.ShapeDtypeStruct((NROWS, 4, W), jnp.int32),
            jax.ShapeDtypeStruct((NROWS, 16, W), jnp.float32),
        ],
    )(scale_arr, frame1, depth1, flow12)


def _compose_body(canvas_ref, frame2_ref, out_ref, mask_ref):
    cv = canvas_ref[...]
    cv = cv[:, :, :, 1:1921]
    rgb = cv[:, 0:3]
    wsum = cv[:, 3:4]
    m = wsum > 0.0
    safe = jnp.where(m, wsum, 1.0)
    warped = jnp.clip(jnp.where(m, rgb / safe, -1.0), -1.0, 1.0)
    mf = m.astype(jnp.float32)
    out_ref[...] = mf * warped + (1.0 - mf) * frame2_ref[...]
    mask_ref[...] = mf


def _compose(canvas_planar, true_frame2):
    return pl.pallas_call(
        _compose_body,
        grid=(B, H // ROWS_BLK),
        in_specs=[
            pl.BlockSpec((1, 4, ROWS_BLK, WP), lambda b, i: (b, 0, i + 1, 0)),
            pl.BlockSpec((1, 3, ROWS_BLK, W), lambda b, i: (b, 0, i, 0)),
        ],
        out_specs=[
            pl.BlockSpec((1, 3, ROWS_BLK, W), lambda b, i: (b, 0, i, 0)),
            pl.BlockSpec((1, 1, ROWS_BLK, W), lambda b, i: (b, 0, i, 0)),
        ],
        out_shape=[
            jax.ShapeDtypeStruct((B, C, H, W), jnp.float32),
            jax.ShapeDtypeStruct((B, 1, H, W), jnp.float32),
        ],
    )(canvas_planar, true_frame2)


def kernel(frame1, frame2, depth1, flow12):
    blkmax = _logmax(depth1)
    scale = 50.0 / jnp.max(blkmax)
    scale_arr = jnp.full((1, 1), scale, jnp.float32)
    idx, vals = _records(scale_arr, frame1, depth1, flow12)
    idx4 = idx.transpose(0, 2, 1).reshape(-1)
    vals4 = vals.transpose(0, 2, 1).reshape(-1, 4, 4).reshape(-1, 4)
    canvas = jnp.zeros((NCELL, 4), jnp.float32).at[idx4].add(vals4)
    planar = canvas.reshape(B, HP, WP, 4).transpose(0, 3, 1, 2)
    return _compose(planar, frame2[0])
